# baseline (speedup 1.0000x reference)
import functools

import jax
import jax.numpy as jnp
from jax import lax
from jax.experimental import pallas as pl
from jax.experimental.pallas import tpu as pltpu
from jax.experimental.pallas import tpu_sc as plsc

NUM_USERS = 50000
NUM_SONGS = 100000
NUM_ARTISTS = 10000
EMB = 64
HID = 64
HEADS = 4
OUT = HID // HEADS
LAYERS = 2
ROW = 80
SENTINEL = 1 << 30

NP = {"user": 50176, "song": 100352, "artist": 10240}
CK = {"user": (4, 12544), "song": (8, 12544), "artist": (2, 5120)}
NREAL = {"user": NUM_USERS, "song": NUM_SONGS, "artist": NUM_ARTISTS}


def _mm_body(x_ref, w_ref, b_ref, o_ref):
    o_ref[...] = jnp.dot(x_ref[...], w_ref[...],
                         preferred_element_type=jnp.float32) + b_ref[...]


def _mm(x, w, b, block=512):
    n, d = x.shape
    dout = w.shape[1]
    return pl.pallas_call(
        _mm_body,
        grid=(n // block,),
        in_specs=[
            pl.BlockSpec((block, d), lambda i: (i, 0)),
            pl.BlockSpec((d, dout), lambda i: (0, 0)),
            pl.BlockSpec((dout,), lambda i: (0,)),
        ],
        out_specs=pl.BlockSpec((block, dout), lambda i: (i, 0)),
        out_shape=jax.ShapeDtypeStruct((n, dout), jnp.float32),
    )(x, w, b)


def _tab_body(x_ref, w_ref, a_ref, o_ref):
    p = jnp.dot(x_ref[...], w_ref[...], preferred_element_type=jnp.float32)
    att = jnp.dot(p, a_ref[...], preferred_element_type=jnp.float32)
    z = jnp.zeros((p.shape[0], ROW - HID - HEADS), jnp.float32)
    o_ref[...] = jnp.concatenate([p, att, z], axis=1)


def _tab(x, w, amat, block=512):
    n, d = x.shape
    return pl.pallas_call(
        _tab_body,
        grid=(n // block,),
        in_specs=[
            pl.BlockSpec((block, d), lambda i: (i, 0)),
            pl.BlockSpec((d, HID), lambda i: (0, 0)),
            pl.BlockSpec((HID, HEADS), lambda i: (0, 0)),
        ],
        out_specs=pl.BlockSpec((block, ROW), lambda i: (i, 0)),
        out_shape=jax.ShapeDtypeStruct((n, ROW), jnp.float32),
    )(x, w, amat)


ADW = 16


def _ad_body(x_ref, w_ref, a_ref, o_ref):
    p = jnp.dot(x_ref[...], w_ref[...], preferred_element_type=jnp.float32)
    o_ref[...] = jnp.dot(p, a_ref[...], preferred_element_type=jnp.float32)


def _ad(x, w, amat, block=512):
    n, d = x.shape
    amat16 = jnp.pad(amat, ((0, 0), (0, ADW - HEADS)))
    return pl.pallas_call(
        _ad_body,
        grid=(n // block,),
        in_specs=[
            pl.BlockSpec((block, d), lambda i: (i, 0)),
            pl.BlockSpec((d, HID), lambda i: (0, 0)),
            pl.BlockSpec((HID, ADW), lambda i: (0, 0)),
        ],
        out_specs=pl.BlockSpec((block, ADW), lambda i: (i, 0)),
        out_shape=jax.ShapeDtypeStruct((n, ADW), jnp.float32),
    )(x, w, amat16)


def _combine_body(two, resid, a1_ref, b1_ref, a2_ref, b2_ref, hp_ref,
                  g_ref, bb_ref, o_ref):
    rmat = jnp.repeat(jnp.eye(HEADS, dtype=jnp.float32), OUT, axis=1)

    def term(a_ref, b_ref):
        a = a_ref[...]
        num = a[:, :HID]
        den = jnp.dot(a[:, HID:HID + HEADS], rmat,
                      preferred_element_type=jnp.float32)
        return num / (den + 1e-16) + b_ref[...]

    y = term(a1_ref, b1_ref)
    if two:
        y = y + term(a2_ref, b2_ref)
    y = jnp.maximum(y, 0.0)
    if resid:
        y = y + hp_ref[...]
    mu = jnp.mean(y, axis=-1, keepdims=True)
    var = jnp.mean((y - mu) ** 2, axis=-1, keepdims=True)
    o_ref[...] = (y - mu) / jnp.sqrt(var + 1e-5) * g_ref[...] + bb_ref[...]


def _combine(a1, b1, a2, b2, hp, lng, lnb, resid, block=512):
    n = a1.shape[0]
    two = a2 is not None
    if a2 is None:
        a2, b2 = a1, b1
    body = functools.partial(_combine_body, two, resid)
    return pl.pallas_call(
        body,
        grid=(n // block,),
        in_specs=[
            pl.BlockSpec((block, ROW), lambda i: (i, 0)),
            pl.BlockSpec((HID,), lambda i: (0,)),
            pl.BlockSpec((block, ROW), lambda i: (i, 0)),
            pl.BlockSpec((HID,), lambda i: (0,)),
            pl.BlockSpec((block, HID), lambda i: (i, 0)),
            pl.BlockSpec((HID,), lambda i: (0,)),
            pl.BlockSpec((HID,), lambda i: (0,)),
        ],
        out_specs=pl.BlockSpec((block, HID), lambda i: (i, 0)),
        out_shape=jax.ShapeDtypeStruct((n, HID), jnp.float32),
    )(a1, b1, a2, b2, hp, lng, lnb)


_B = 4096
_G = 128
_PEND = _B + 256
_ZV = 32
_EV = 128


def _pieces(rpt, step):
    out, o = [], 0
    while o < rpt:
        s = min(step, rpt - o)
        out.append((o, s))
        o += s
    return out


def _edge_body(C, K, E_pad,
               tab, ad, srcE, dstE, out,
               acc, dstb, srcb, pend_d, pend_s,
               gidx0, gidx1, aidx0, aidx1, sidx0, sidx1,
               rowbuf0, rowbuf1, adbuf0, adbuf1, msgbuf0, msgbuf1, zbuf,
               exbuf, sem0, sem1, ssem0, ssem1):
    cid = lax.axis_index("c")
    sid = lax.axis_index("s")
    C_sc = C // 2
    L = E_pad // 16
    RPT = K // 16
    lane = lax.iota(jnp.int32, 16)
    zv = jnp.zeros((16,), jnp.float32)
    zi = jnp.zeros((16,), jnp.int32)

    def fillz(r, car2):
        for k in range(ROW // 16):
            zbuf[r, pl.ds(k * 16, 16)] = zv
        return car2
    lax.fori_loop(0, _ZV, fillz, 0)

    def initp(i, car):
        pend_d[pl.ds(i * 16, 16)] = zi
        pend_s[pl.ds(i * 16, 16)] = zi
        return car
    lax.fori_loop(0, _PEND // 16, initp, 0)

    gbufs = ((gidx0, aidx0, rowbuf0, adbuf0, sem0, sidx0, msgbuf0, ssem0),
             (gidx1, aidx1, rowbuf1, adbuf1, sem1, sidx1, msgbuf1, ssem1))

    def chunk_body(j, car):
        c = cid * C_sc + j
        for (o, s) in _pieces(RPT, _ZV):
            pltpu.sync_copy(zbuf.at[pl.ds(0, s)],
                            acc.at[pl.ds(sid * RPT + o, s)])
        plsc.subcore_barrier()

        def block_body(b, car2):
            base_e = sid * L + b * _B
            pltpu.sync_copy(dstE.at[pl.ds(base_e, _B)], dstb)
            pltpu.sync_copy(srcE.at[pl.ds(base_e, _B)], srcb)

            def scan_g(g, ptr):
                dv = dstb[pl.ds(g * 16, 16)]
                sv = srcb[pl.ds(g * 16, 16)]
                dl = dv - c * K
                m = (dl >= 0) & (dl < K)
                cs = plsc.cumsum(m.astype(jnp.int32))
                pos = ptr + cs - 1
                plsc.store_scatter(pend_d, [pos], dl, mask=m)
                plsc.store_scatter(pend_s, [pos], sv, mask=m)
                return ptr + cs[15]
            n = lax.fori_loop(0, _B // 16, scan_g, jnp.int32(0), unroll=4)

            def issue(q, slot):
                g_ref, a_ref, r_ref, ab_ref, s_ref = gbufs[slot][:5]
                for t in range(_G // 16):
                    g_ref[pl.ds(t * 16, 16)] = pend_s[
                        pl.ds(q * _G + t * 16, 16)]
                    a_ref[pl.ds(t * 16, 16)] = pend_d[
                        pl.ds(q * _G + t * 16, 16)] + c * K
                pltpu.async_copy(tab.at[g_ref], r_ref, s_ref)
                pltpu.async_copy(ad.at[a_ref], ab_ref, s_ref)

            def process(q, p, slot):
                (g_ref, a_ref, r_ref, ab_ref, s_ref,
                 si_ref, mb_ref, ss_ref) = gbufs[slot]
                pltpu.make_async_copy(tab.at[g_ref], r_ref, s_ref).wait()
                pltpu.make_async_copy(ad.at[a_ref], ab_ref, s_ref).wait()

                @pl.when(p > 0)
                def _():
                    pltpu.make_async_copy(mb_ref, acc.at[si_ref],
                                          ss_ref).wait()
                base = q * _G
                for t in range(_G // 16):
                    si_ref[pl.ds(t * 16, 16)] = pend_d[
                        pl.ds(base + t * 16, 16)]
                for g in range(_G // 16):
                    valid = (base + g * 16 + lane) < n
                    rows = g * 16 + lane
                    for h in range(HEADS):
                        a_s = plsc.load_gather(
                            r_ref, [rows,
                                    jnp.full((16,), HID + h, jnp.int32)])
                        a_d = plsc.load_gather(
                            ab_ref, [rows, jnp.full((16,), h, jnp.int32)])
                        al = a_s + a_d
                        al = jnp.where(al >= 0.0, al, 0.2 * al)
                        ex = jnp.where(valid, jnp.exp(al), 0.0)
                        exbuf[pl.ds(h * 16, 16)] = ex
                        plsc.store_scatter(
                            mb_ref, [rows, jnp.full((16,), HID + h,
                                                    jnp.int32)], ex)
                    for h in range(HEADS):
                        exv_h = exbuf[pl.ds(h * 16, 16)]
                        for e in range(16):
                            r = g * 16 + e
                            mb_ref[r, pl.ds(h * 16, 16)] = (
                                r_ref[r, pl.ds(h * 16, 16)] * exv_h[e])
                pltpu.async_copy(mb_ref, acc.at[si_ref], ss_ref, add=True)

            npair = (n + 2 * _G - 1) // (2 * _G)

            @pl.when(n > 0)
            def _flush():
                issue(0, 0)

                def pair(p, car3):
                    issue(2 * p + 1, 1)
                    process(2 * p, p, 0)

                    @pl.when(p + 1 < npair)
                    def _():
                        issue(2 * p + 2, 0)
                    process(2 * p + 1, p, 1)
                    return car3
                lax.fori_loop(0, npair, pair, 0)
                pltpu.make_async_copy(msgbuf0, acc.at[sidx0], ssem0).wait()
                pltpu.make_async_copy(msgbuf1, acc.at[sidx1], ssem1).wait()
            return car2
        lax.fori_loop(0, L // _B, block_body, 0)
        plsc.subcore_barrier()
        for (o, s) in _pieces(RPT, _EV):
            pltpu.sync_copy(acc.at[pl.ds(sid * RPT + o, s)],
                            msgbuf0.at[pl.ds(0, s)])
            pltpu.sync_copy(msgbuf0.at[pl.ds(0, s)],
                            out.at[pl.ds(c * K + sid * RPT + o, s)])
        plsc.subcore_barrier()
        return car
    lax.fori_loop(0, C_sc, chunk_body, 0)


def _edge_sc(tab, ad2d, srcE, dstE, np_dst, C, K, E_pad):
    body = functools.partial(_edge_body, C, K, E_pad)
    mesh = plsc.VectorSubcoreMesh(core_axis_name="c", subcore_axis_name="s")
    f = pl.kernel(
        body,
        out_type=jax.ShapeDtypeStruct((np_dst, ROW), jnp.float32),
        mesh=mesh,
        compiler_params=pltpu.CompilerParams(use_tc_tiling_on_sc=False,
                                             needs_layout_passes=False),
        scratch_types=[
            pltpu.VMEM_SHARED((K, ROW), jnp.float32),
            pltpu.VMEM((_B,), jnp.int32),
            pltpu.VMEM((_B,), jnp.int32),
            pltpu.VMEM((_PEND,), jnp.int32),
            pltpu.VMEM((_PEND,), jnp.int32),
            pltpu.VMEM((_G,), jnp.int32),
            pltpu.VMEM((_G,), jnp.int32),
            pltpu.VMEM((_G,), jnp.int32),
            pltpu.VMEM((_G,), jnp.int32),
            pltpu.VMEM((_G,), jnp.int32),
            pltpu.VMEM((_G,), jnp.int32),
            pltpu.VMEM((_G, ROW), jnp.float32),
            pltpu.VMEM((_G, ROW), jnp.float32),
            pltpu.VMEM((_G, ADW), jnp.float32),
            pltpu.VMEM((_G, ADW), jnp.float32),
            pltpu.VMEM((_G, ROW), jnp.float32),
            pltpu.VMEM((_G, ROW), jnp.float32),
            pltpu.VMEM((_ZV, ROW), jnp.float32),
            pltpu.VMEM((HEADS * 16,), jnp.float32),
            pltpu.SemaphoreType.DMA,
            pltpu.SemaphoreType.DMA,
            pltpu.SemaphoreType.DMA,
            pltpu.SemaphoreType.DMA,
        ],
    )
    return f(tab, ad2d, srcE, dstE)


def _pad_rows(x, n):
    return jnp.pad(x, ((0, n - x.shape[0]), (0, 0)))


def _pad_edges(src, dst, e_pad):
    n = src.shape[0]
    return (jnp.concatenate([src, jnp.zeros((e_pad - n,), jnp.int32)]),
            jnp.concatenate([dst, jnp.full((e_pad - n,), SENTINEL,
                                           jnp.int32)]))


def _amat(att):
    return jnp.repeat(jnp.eye(HEADS, dtype=jnp.float32), OUT,
                      axis=0) * att.reshape(-1, 1)


def kernel(params, x_user, x_song, x_artist, ls_src, ls_dst, by_src, by_dst):
    h = {}
    for t, emb in [("user", params["emb_user"]), ("song", params["emb_song"]),
                   ("artist", params["emb_artist"])]:
        h[t] = _mm(_pad_rows(emb, NP[t]), params["in_proj"][t]["W"],
                   params["in_proj"][t]["b"])

    e_ls = 1 << 20
    e_by = 1 << 17
    rl_s, rl_d = _pad_edges(ls_dst, ls_src, e_ls)
    ls_s, ls_d = _pad_edges(ls_src, ls_dst, e_ls)
    rb_s, rb_d = _pad_edges(by_dst, by_src, e_by)
    by_s, by_d = _pad_edges(by_src, by_dst, e_by)
    convs = [("rl", "song", "user", rl_s, rl_d, e_ls),
             ("ls", "user", "song", ls_s, ls_d, e_ls),
             ("rb", "artist", "song", rb_s, rb_d, e_by),
             ("by", "song", "artist", by_s, by_d, e_by)]

    for i in range(LAYERS):
        g = params["gat"][i]
        acc = {}
        for name, st, dt, se, de, ep in convs:
            p = g[name]
            tab = _tab(h[st], p["W"], _amat(p["att_src"]))
            advec = _ad(h[dt], p["W"], _amat(p["att_dst"]))
            C, K = CK[dt]
            acc[name] = _edge_sc(tab, advec, se, de, NP[dt], C, K, ep)
        ln = params["ln"][i]
        resid = i > 0
        h = {
            "user": _combine(acc["rl"], g["rl"]["b"], None, None,
                             h["user"], ln["g"], ln["b"], resid),
            "song": _combine(acc["ls"], g["ls"]["b"], acc["rb"], g["rb"]["b"],
                             h["song"], ln["g"], ln["b"], resid),
            "artist": _combine(acc["by"], g["by"]["b"], None, None,
                               h["artist"], ln["g"], ln["b"], resid),
        }

    outs = {t: _mm(h[t], params["out_proj"][t]["W"],
                   params["out_proj"][t]["b"])[:NREAL[t]]
            for t in ("user", "song", "artist")}
    return (outs["user"], outs["song"], outs["artist"])


# speedup vs baseline: 1.5144x; 1.5144x over previous
import functools

import jax
import jax.numpy as jnp
from jax import lax
from jax.experimental import pallas as pl
from jax.experimental.pallas import tpu as pltpu
from jax.experimental.pallas import tpu_sc as plsc

NUM_USERS = 50000
NUM_SONGS = 100000
NUM_ARTISTS = 10000
EMB = 64
HID = 64
HEADS = 4
OUT = HID // HEADS
LAYERS = 2
ROW = 80
TROW = 36
SENTINEL = 1 << 30

NP = {"user": 50176, "song": 100352, "artist": 10240}
CK = {"user": (4, 12544), "song": (8, 12544), "artist": (2, 5120)}
NREAL = {"user": NUM_USERS, "song": NUM_SONGS, "artist": NUM_ARTISTS}


def _mm_body(x_ref, w_ref, b_ref, o_ref):
    o_ref[...] = jnp.dot(x_ref[...], w_ref[...],
                         preferred_element_type=jnp.float32) + b_ref[...]


def _mm(x, w, b, block=512):
    n, d = x.shape
    dout = w.shape[1]
    return pl.pallas_call(
        _mm_body,
        grid=(n // block,),
        in_specs=[
            pl.BlockSpec((block, d), lambda i: (i, 0)),
            pl.BlockSpec((d, dout), lambda i: (0, 0)),
            pl.BlockSpec((dout,), lambda i: (0,)),
        ],
        out_specs=pl.BlockSpec((block, dout), lambda i: (i, 0)),
        out_shape=jax.ShapeDtypeStruct((n, dout), jnp.float32),
    )(x, w, b)


def _tab_body(x_ref, w_ref, a_ref, o_ref):
    p = jnp.dot(x_ref[...], w_ref[...], preferred_element_type=jnp.float32)
    att = jnp.dot(p, a_ref[...], preferred_element_type=jnp.float32)
    pb = jax.lax.bitcast_convert_type(p.astype(jnp.bfloat16), jnp.uint16)
    lo = jnp.concatenate([pb[:, 0:16], pb[:, 32:48]], axis=1)
    hi = jnp.concatenate([pb[:, 16:32], pb[:, 48:64]], axis=1)
    packed = lo.astype(jnp.uint32) | (hi.astype(jnp.uint32) << 16)
    packedf = jax.lax.bitcast_convert_type(packed, jnp.float32)
    o_ref[...] = jnp.concatenate([packedf, att], axis=1)


def _tab(x, w, amat, block=512):
    n, d = x.shape
    return pl.pallas_call(
        _tab_body,
        grid=(n // block,),
        in_specs=[
            pl.BlockSpec((block, d), lambda i: (i, 0)),
            pl.BlockSpec((d, HID), lambda i: (0, 0)),
            pl.BlockSpec((HID, HEADS), lambda i: (0, 0)),
        ],
        out_specs=pl.BlockSpec((block, TROW), lambda i: (i, 0)),
        out_shape=jax.ShapeDtypeStruct((n, TROW), jnp.float32),
    )(x, w, amat)


ADW = 16


def _ad_body(x_ref, w_ref, a_ref, o_ref):
    p = jnp.dot(x_ref[...], w_ref[...], preferred_element_type=jnp.float32)
    o_ref[...] = jnp.dot(p, a_ref[...], preferred_element_type=jnp.float32)


def _ad(x, w, amat, block=512):
    n, d = x.shape
    amat16 = jnp.pad(amat, ((0, 0), (0, ADW - HEADS)))
    return pl.pallas_call(
        _ad_body,
        grid=(n // block,),
        in_specs=[
            pl.BlockSpec((block, d), lambda i: (i, 0)),
            pl.BlockSpec((d, HID), lambda i: (0, 0)),
            pl.BlockSpec((HID, ADW), lambda i: (0, 0)),
        ],
        out_specs=pl.BlockSpec((block, ADW), lambda i: (i, 0)),
        out_shape=jax.ShapeDtypeStruct((n, ADW), jnp.float32),
    )(x, w, amat16)


def _combine_body(two, resid, a1_ref, b1_ref, a2_ref, b2_ref, hp_ref,
                  g_ref, bb_ref, o_ref):
    rmat = jnp.repeat(jnp.eye(HEADS, dtype=jnp.float32), OUT, axis=1)

    def term(a_ref, b_ref):
        a = a_ref[...]
        num = a[:, :HID]
        den = jnp.dot(a[:, HID:HID + HEADS], rmat,
                      preferred_element_type=jnp.float32)
        return num / (den + 1e-16) + b_ref[...]

    y = term(a1_ref, b1_ref)
    if two:
        y = y + term(a2_ref, b2_ref)
    y = jnp.maximum(y, 0.0)
    if resid:
        y = y + hp_ref[...]
    mu = jnp.mean(y, axis=-1, keepdims=True)
    var = jnp.mean((y - mu) ** 2, axis=-1, keepdims=True)
    o_ref[...] = (y - mu) / jnp.sqrt(var + 1e-5) * g_ref[...] + bb_ref[...]


def _combine(a1, b1, a2, b2, hp, lng, lnb, resid, block=512):
    n = a1.shape[0]
    two = a2 is not None
    if a2 is None:
        a2, b2 = a1, b1
    body = functools.partial(_combine_body, two, resid)
    return pl.pallas_call(
        body,
        grid=(n // block,),
        in_specs=[
            pl.BlockSpec((block, ROW), lambda i: (i, 0)),
            pl.BlockSpec((HID,), lambda i: (0,)),
            pl.BlockSpec((block, ROW), lambda i: (i, 0)),
            pl.BlockSpec((HID,), lambda i: (0,)),
            pl.BlockSpec((block, HID), lambda i: (i, 0)),
            pl.BlockSpec((HID,), lambda i: (0,)),
            pl.BlockSpec((HID,), lambda i: (0,)),
        ],
        out_specs=pl.BlockSpec((block, HID), lambda i: (i, 0)),
        out_shape=jax.ShapeDtypeStruct((n, HID), jnp.float32),
    )(a1, b1, a2, b2, hp, lng, lnb)


_B = 4096
_G = 128
_PEND = _B + 256
_ZV = 32
_EV = 128


def _pieces(rpt, step):
    out, o = [], 0
    while o < rpt:
        s = min(step, rpt - o)
        out.append((o, s))
        o += s
    return out


def _edge_body(C, K, E_pad,
               tab, ad, srcE, dstE, out,
               acc, dstb, srcb, pend_d, pend_s,
               gidx0, gidx1, aidx0, aidx1, sidx0, sidx1,
               rowbuf0, rowbuf1, adbuf0, adbuf1, msgbuf0, msgbuf1, zbuf,
               exbuf, sem0, sem1, ssem0, ssem1):
    cid = lax.axis_index("c")
    sid = lax.axis_index("s")
    C_sc = C // 2
    L = E_pad // 16
    RPT = K // 16
    lane = lax.iota(jnp.int32, 16)
    zv = jnp.zeros((16,), jnp.float32)
    zi = jnp.zeros((16,), jnp.int32)

    def fillz(r, car2):
        for k in range(ROW // 16):
            zbuf[r, pl.ds(k * 16, 16)] = zv
        return car2
    lax.fori_loop(0, _ZV, fillz, 0)

    def initp(i, car):
        pend_d[pl.ds(i * 16, 16)] = zi
        pend_s[pl.ds(i * 16, 16)] = zi
        return car
    lax.fori_loop(0, _PEND // 16, initp, 0)

    gbufs = ((gidx0, aidx0, rowbuf0, adbuf0, sem0, sidx0, msgbuf0, ssem0),
             (gidx1, aidx1, rowbuf1, adbuf1, sem1, sidx1, msgbuf1, ssem1))

    def chunk_body(j, car):
        c = cid * C_sc + j
        for (o, s) in _pieces(RPT, _ZV):
            pltpu.sync_copy(zbuf.at[pl.ds(0, s)],
                            acc.at[pl.ds(sid * RPT + o, s)])
        plsc.subcore_barrier()

        def block_body(b, car2):
            base_e = sid * L + b * _B
            pltpu.sync_copy(dstE.at[pl.ds(base_e, _B)], dstb)
            pltpu.sync_copy(srcE.at[pl.ds(base_e, _B)], srcb)

            def scan_g(g, ptr):
                dv = dstb[pl.ds(g * 16, 16)]
                sv = srcb[pl.ds(g * 16, 16)]
                dl = dv - c * K
                m = (dl >= 0) & (dl < K)
                cs = plsc.cumsum(m.astype(jnp.int32))
                pos = ptr + cs - 1
                plsc.store_scatter(pend_d, [pos], dl, mask=m)
                plsc.store_scatter(pend_s, [pos], sv, mask=m)
                return ptr + cs[15]
            n = lax.fori_loop(0, _B // 16, scan_g, jnp.int32(0), unroll=4)

            def issue(q, slot):
                g_ref, a_ref, r_ref, ab_ref, s_ref = gbufs[slot][:5]
                for t in range(_G // 16):
                    g_ref[pl.ds(t * 16, 16)] = pend_s[
                        pl.ds(q * _G + t * 16, 16)]
                    a_ref[pl.ds(t * 16, 16)] = pend_d[
                        pl.ds(q * _G + t * 16, 16)] + c * K
                pltpu.async_copy(tab.at[g_ref], r_ref, s_ref)
                pltpu.async_copy(ad.at[a_ref], ab_ref, s_ref)

            def process(q, p, slot):
                (g_ref, a_ref, r_ref, ab_ref, s_ref,
                 si_ref, mb_ref, ss_ref) = gbufs[slot]
                pltpu.make_async_copy(tab.at[g_ref], r_ref, s_ref).wait()
                pltpu.make_async_copy(ad.at[a_ref], ab_ref, s_ref).wait()

                @pl.when(p > 0)
                def _():
                    pltpu.make_async_copy(mb_ref, acc.at[si_ref],
                                          ss_ref).wait()
                base = q * _G
                for t in range(_G // 16):
                    si_ref[pl.ds(t * 16, 16)] = pend_d[
                        pl.ds(base + t * 16, 16)]
                for g in range(_G // 16):
                    valid = (base + g * 16 + lane) < n
                    rows = g * 16 + lane
                    for h in range(HEADS):
                        a_s = plsc.load_gather(
                            r_ref, [rows,
                                    jnp.full((16,), 2 * HID // 4 + h,
                                             jnp.int32)])
                        a_d = plsc.load_gather(
                            ab_ref, [rows, jnp.full((16,), h, jnp.int32)])
                        al = a_s + a_d
                        al = jnp.where(al >= 0.0, al, 0.2 * al)
                        ex = jnp.where(valid, jnp.exp(al), 0.0)
                        exbuf[pl.ds(h * 16, 16)] = ex
                        plsc.store_scatter(
                            mb_ref, [rows, jnp.full((16,), HID + h,
                                                    jnp.int32)], ex)
                    for hp in range(HEADS // 2):
                        ex_lo = exbuf[pl.ds(2 * hp * 16, 16)]
                        ex_hi = exbuf[pl.ds((2 * hp + 1) * 16, 16)]
                        for e in range(16):
                            r = g * 16 + e
                            w = plsc.bitcast(
                                r_ref[r, pl.ds(hp * 16, 16)], jnp.uint32)
                            vlo = plsc.bitcast(w << 16, jnp.float32)
                            vhi = plsc.bitcast((w >> 16) << 16, jnp.float32)
                            mb_ref[r, pl.ds(2 * hp * 16, 16)] = (
                                vlo * ex_lo[e])
                            mb_ref[r, pl.ds((2 * hp + 1) * 16, 16)] = (
                                vhi * ex_hi[e])
                pltpu.async_copy(mb_ref, acc.at[si_ref], ss_ref, add=True)

            npair = (n + 2 * _G - 1) // (2 * _G)

            @pl.when(n > 0)
            def _flush():
                issue(0, 0)

                def pair(p, car3):
                    issue(2 * p + 1, 1)
                    process(2 * p, p, 0)

                    @pl.when(p + 1 < npair)
                    def _():
                        issue(2 * p + 2, 0)
                    process(2 * p + 1, p, 1)
                    return car3
                lax.fori_loop(0, npair, pair, 0)
                pltpu.make_async_copy(msgbuf0, acc.at[sidx0], ssem0).wait()
                pltpu.make_async_copy(msgbuf1, acc.at[sidx1], ssem1).wait()
            return car2
        lax.fori_loop(0, L // _B, block_body, 0)
        plsc.subcore_barrier()
        for (o, s) in _pieces(RPT, _EV):
            pltpu.sync_copy(acc.at[pl.ds(sid * RPT + o, s)],
                            msgbuf0.at[pl.ds(0, s)])
            pltpu.sync_copy(msgbuf0.at[pl.ds(0, s)],
                            out.at[pl.ds(c * K + sid * RPT + o, s)])
        plsc.subcore_barrier()
        return car
    lax.fori_loop(0, C_sc, chunk_body, 0)


def _edge_sc(tab, ad2d, srcE, dstE, np_dst, C, K, E_pad):
    body = functools.partial(_edge_body, C, K, E_pad)
    mesh = plsc.VectorSubcoreMesh(core_axis_name="c", subcore_axis_name="s")
    f = pl.kernel(
        body,
        out_type=jax.ShapeDtypeStruct((np_dst, ROW), jnp.float32),
        mesh=mesh,
        compiler_params=pltpu.CompilerParams(use_tc_tiling_on_sc=False,
                                             needs_layout_passes=False),
        scratch_types=[
            pltpu.VMEM_SHARED((K, ROW), jnp.float32),
            pltpu.VMEM((_B,), jnp.int32),
            pltpu.VMEM((_B,), jnp.int32),
            pltpu.VMEM((_PEND,), jnp.int32),
            pltpu.VMEM((_PEND,), jnp.int32),
            pltpu.VMEM((_G,), jnp.int32),
            pltpu.VMEM((_G,), jnp.int32),
            pltpu.VMEM((_G,), jnp.int32),
            pltpu.VMEM((_G,), jnp.int32),
            pltpu.VMEM((_G,), jnp.int32),
            pltpu.VMEM((_G,), jnp.int32),
            pltpu.VMEM((_G, TROW), jnp.float32),
            pltpu.VMEM((_G, TROW), jnp.float32),
            pltpu.VMEM((_G, ADW), jnp.float32),
            pltpu.VMEM((_G, ADW), jnp.float32),
            pltpu.VMEM((_G, ROW), jnp.float32),
            pltpu.VMEM((_G, ROW), jnp.float32),
            pltpu.VMEM((_ZV, ROW), jnp.float32),
            pltpu.VMEM((HEADS * 16,), jnp.float32),
            pltpu.SemaphoreType.DMA,
            pltpu.SemaphoreType.DMA,
            pltpu.SemaphoreType.DMA,
            pltpu.SemaphoreType.DMA,
        ],
    )
    return f(tab, ad2d, srcE, dstE)


def _pad_rows(x, n):
    return jnp.pad(x, ((0, n - x.shape[0]), (0, 0)))


def _pad_edges(src, dst, e_pad):
    n = src.shape[0]
    return (jnp.concatenate([src, jnp.zeros((e_pad - n,), jnp.int32)]),
            jnp.concatenate([dst, jnp.full((e_pad - n,), SENTINEL,
                                           jnp.int32)]))


def _amat(att):
    return jnp.repeat(jnp.eye(HEADS, dtype=jnp.float32), OUT,
                      axis=0) * att.reshape(-1, 1)


def kernel(params, x_user, x_song, x_artist, ls_src, ls_dst, by_src, by_dst):
    h = {}
    for t, emb in [("user", params["emb_user"]), ("song", params["emb_song"]),
                   ("artist", params["emb_artist"])]:
        h[t] = _mm(_pad_rows(emb, NP[t]), params["in_proj"][t]["W"],
                   params["in_proj"][t]["b"])

    e_ls = 1 << 20
    e_by = 1 << 17
    rl_s, rl_d = _pad_edges(ls_dst, ls_src, e_ls)
    ls_s, ls_d = _pad_edges(ls_src, ls_dst, e_ls)
    rb_s, rb_d = _pad_edges(by_dst, by_src, e_by)
    by_s, by_d = _pad_edges(by_src, by_dst, e_by)
    convs = [("rl", "song", "user", rl_s, rl_d, e_ls),
             ("ls", "user", "song", ls_s, ls_d, e_ls),
             ("rb", "artist", "song", rb_s, rb_d, e_by),
             ("by", "song", "artist", by_s, by_d, e_by)]

    for i in range(LAYERS):
        g = params["gat"][i]
        acc = {}
        for name, st, dt, se, de, ep in convs:
            p = g[name]
            tab = _tab(h[st], p["W"], _amat(p["att_src"]))
            advec = _ad(h[dt], p["W"], _amat(p["att_dst"]))
            C, K = CK[dt]
            acc[name] = _edge_sc(tab, advec, se, de, NP[dt], C, K, ep)
        ln = params["ln"][i]
        resid = i > 0
        h = {
            "user": _combine(acc["rl"], g["rl"]["b"], None, None,
                             h["user"], ln["g"], ln["b"], resid),
            "song": _combine(acc["ls"], g["ls"]["b"], acc["rb"], g["rb"]["b"],
                             h["song"], ln["g"], ln["b"], resid),
            "artist": _combine(acc["by"], g["by"]["b"], None, None,
                               h["artist"], ln["g"], ln["b"], resid),
        }

    outs = {t: _mm(h[t], params["out_proj"][t]["W"],
                   params["out_proj"][t]["b"])[:NREAL[t]]
            for t in ("user", "song", "artist")}
    return (outs["user"], outs["song"], outs["artist"])
